# baseline (speedup 1.0000x reference)
import functools
import jax
import jax.numpy as jnp
from jax import lax
from jax.experimental import pallas as pl
from jax.experimental.pallas import tpu as pltpu
from jax.experimental.pallas import tpu_sc as plsc

NC = 2
NS = 16
NW = NC * NS

BATCH = 16384
HIST = 200
DIM = 64
VOCAB = 1000

HT = HIST // 8
BT = BATCH // 128
BT_PER_W = BT // NW
N_TILES = HT * BT_PER_W


@functools.partial(
    pl.kernel,
    out_type=jax.ShapeDtypeStruct((HIST, DIM, BATCH), jnp.float32),
    mesh=plsc.VectorSubcoreMesh(core_axis_name="c", subcore_axis_name="s"),
    scratch_types=[
        pltpu.VMEM((VOCAB // 2, 128), jnp.float32),
        pltpu.VMEM((2, 8, 128), jnp.int32),
        pltpu.VMEM((2, DIM, 128), jnp.float32),
        pltpu.SemaphoreType.DMA,
        pltpu.SemaphoreType.DMA,
    ],
    compiler_params=pltpu.CompilerParams(needs_layout_passes=False),
)
def _emb_lookup(xt_hbm, tab_hbm, out_hbm, tab_v, idx_v, trans_v, sem_i, sem_o):
    wid = lax.axis_index("s") * NC + lax.axis_index("c")
    c0 = wid * BT_PER_W

    pltpu.sync_copy(tab_hbm, tab_v)

    def idx_slice(k):
        ht = k >> 2
        c = c0 + (k & 3)
        return xt_hbm.at[pl.ds(ht * 8, 8), pl.ds(c * 128, 128)]

    def fire_idx(k, b):
        return pltpu.async_copy(idx_slice(k), idx_v.at[b], sem_i)

    def out_slice(hh, c):
        return out_hbm.at[hh].at[:, pl.ds(c * 128, 128)]

    fire_idx(0, 0)

    def tile_body(k, carry):
        b = k & 1
        pltpu.make_async_copy(idx_slice(k), idx_v.at[b], sem_i).wait()

        @pl.when(k < N_TILES - 1)
        def _():
            fire_idx(k + 1, 1 - b)

        ht = k >> 2
        c = c0 + (k & 3)

        def h_body(hl, carry2):
            tb = hl & 1

            @pl.when(k * 8 + hl >= 2)
            def _():
                pltpu.make_async_copy(
                    trans_v.at[tb], out_slice(0, 0), sem_o
                ).wait()

            @plsc.parallel_loop(0, 8, 1, unroll=4)
            def bg_body(bg):
                idx16 = idx_v[b, hl, pl.ds(bg * 16, 16)]
                r16 = idx16 >> 1
                cb = (idx16 & 1) << 6
                for tc in range(0, DIM, 8):
                    vs = [
                        plsc.load_gather(
                            tab_v,
                            [r16, cb + (((tc + i) - idx16) & (DIM - 1))],
                        )
                        for i in range(8)
                    ]
                    for i in range(8):
                        trans_v[tb, tc + i, pl.ds(bg * 16, 16)] = vs[i]

            pltpu.async_copy(trans_v.at[tb], out_slice(ht * 8 + hl, c), sem_o)
            return carry2

        lax.fori_loop(0, 8, h_body, 0)
        return carry

    lax.fori_loop(0, N_TILES, tile_body, 0)

    pltpu.make_async_copy(trans_v.at[0], out_slice(0, 0), sem_o).wait()
    pltpu.make_async_copy(trans_v.at[1], out_slice(0, 0), sem_o).wait()


def kernel(x, table):
    col = (jnp.arange(DIM)[None, :] + jnp.arange(VOCAB)[:, None]) % DIM
    tabs = jnp.take_along_axis(table, col, axis=1)
    tab2 = tabs.reshape(VOCAB // 2, 128)
    xt = jnp.swapaxes(x.astype(jnp.int32), 0, 1)
    out_t = _emb_lookup(xt, tab2)
    return jnp.transpose(out_t, (2, 0, 1))


# speedup vs baseline: 1.3153x; 1.3153x over previous
import functools
import jax
import jax.numpy as jnp
from jax import lax
from jax.experimental import pallas as pl
from jax.experimental.pallas import tpu as pltpu
from jax.experimental.pallas import tpu_sc as plsc

NC = 2
NS = 16
NW = NC * NS

BATCH = 16384
HIST = 200
DIM = 64
VOCAB = 1000

HT = HIST // 8
BT = BATCH // 128
BT_PER_W = BT // NW
N_TILES = HT * BT_PER_W


@functools.partial(
    pl.kernel,
    out_type=jax.ShapeDtypeStruct((HIST, DIM, BATCH), jnp.float32),
    mesh=plsc.VectorSubcoreMesh(core_axis_name="c", subcore_axis_name="s"),
    scratch_types=[
        pltpu.VMEM((VOCAB // 2, 128), jnp.float32),
        pltpu.VMEM((2, 8, 128), jnp.int32),
        pltpu.VMEM((2, DIM, 128), jnp.float32),
        pltpu.SemaphoreType.DMA,
        pltpu.SemaphoreType.DMA,
    ],
    compiler_params=pltpu.CompilerParams(needs_layout_passes=False),
)
def _emb_lookup(xt_hbm, tab_hbm, out_hbm, tab_v, idx_v, trans_v, sem_i, sem_o):
    wid = lax.axis_index("s") * NC + lax.axis_index("c")
    c0 = wid * BT_PER_W

    pltpu.sync_copy(tab_hbm, tab_v)

    def idx_slice(k):
        ht = k >> 2
        c = c0 + (k & 3)
        return xt_hbm.at[pl.ds(ht * 8, 8), pl.ds(c * 128, 128)]

    def fire_idx(k, b):
        return pltpu.async_copy(idx_slice(k), idx_v.at[b], sem_i)

    def out_slice(hh, c):
        return out_hbm.at[hh].at[:, pl.ds(c * 128, 128)]

    fire_idx(0, 0)

    def tile_body(k, carry):
        b = k & 1
        pltpu.make_async_copy(idx_slice(k), idx_v.at[b], sem_i).wait()

        @pl.when(k < N_TILES - 1)
        def _():
            fire_idx(k + 1, 1 - b)

        ht = k >> 2
        c = c0 + (k & 3)

        def h_body(hl, carry2):
            tb = hl & 1

            @pl.when(k * 8 + hl >= 2)
            def _():
                pltpu.make_async_copy(
                    trans_v.at[tb], out_slice(0, 0), sem_o
                ).wait()

            @plsc.parallel_loop(0, 8, 1, unroll=2)
            def bg_body(bg):
                idx16 = idx_v[b, hl, pl.ds(bg * 16, 16)]
                r16 = idx16 >> 1
                cb = (idx16 & 1) << 6
                for tc in range(0, DIM, 8):
                    vs = [
                        plsc.load_gather(
                            tab_v,
                            [r16, cb + (((tc + i) - idx16) & (DIM - 1))],
                        )
                        for i in range(8)
                    ]
                    for i in range(8):
                        trans_v[tb, tc + i, pl.ds(bg * 16, 16)] = vs[i]

            pltpu.async_copy(trans_v.at[tb], out_slice(ht * 8 + hl, c), sem_o)
            return carry2

        lax.fori_loop(0, 8, h_body, 0)
        return carry

    lax.fori_loop(0, N_TILES, tile_body, 0)

    pltpu.make_async_copy(trans_v.at[0], out_slice(0, 0), sem_o).wait()
    pltpu.make_async_copy(trans_v.at[1], out_slice(0, 0), sem_o).wait()


def kernel(x, table):
    col = (jnp.arange(DIM)[None, :] + jnp.arange(VOCAB)[:, None]) % DIM
    tabs = jnp.take_along_axis(table, col, axis=1)
    tab2 = tabs.reshape(VOCAB // 2, 128)
    xt = jnp.swapaxes(x.astype(jnp.int32), 0, 1)
    out_t = _emb_lookup(xt, tab2)
    return jnp.transpose(out_t, (2, 0, 1))
